# baseline (speedup 1.0000x reference)
import functools

import jax
import jax.numpy as jnp
from jax import lax
from jax.experimental import pallas as pl
from jax.experimental.pallas import tpu as pltpu
from jax.experimental.pallas import tpu_sc as plsc

_B, _T, _D = 4, 2048, 1024
_E = 8
_CAP_F = 1.25
_Z_COEFF = 0.001
_N_TOK = _B * _T
_BLK = 2048
_GRID = _N_TOK // _BLK
_CAPACITY = float(int(_CAP_F * _N_TOK / _E))

_NW = 32
_ZTOT = _N_TOK * _D
_ZPW = _ZTOT // _NW
_ZCH = 16384
_ZNC = _ZPW // _ZCH


def _sc_zeros_body(out_hbm, buf, sem):
    c = lax.axis_index("c")
    s = lax.axis_index("s")
    wid = s * 2 + c
    base = wid * _ZPW

    def _zinit(j, carry):
        buf[pl.ds(j * 16, 16)] = jnp.zeros((16,), jnp.float32)
        return carry

    lax.fori_loop(0, _ZCH // 16, _zinit, 0)
    copies = [
        pltpu.async_copy(buf, out_hbm.at[pl.ds(base + k * _ZCH, _ZCH)], sem)
        for k in range(_ZNC)
    ]
    for cp in copies:
        cp.wait()


_sc_zeros = functools.partial(
    pl.kernel,
    mesh=plsc.VectorSubcoreMesh(core_axis_name="c", subcore_axis_name="s"),
    out_type=jax.ShapeDtypeStruct((_ZTOT,), jnp.float32),
    scratch_types=[
        pltpu.VMEM((_ZCH,), jnp.float32),
        pltpu.SemaphoreType.DMA,
    ],
)(_sc_zeros_body)


def _router_body(x_ref, wrt_ref, br_ref, counts_ref, loss_ref):
    i = pl.program_id(0)

    @pl.when(i == 0)
    def _init():
        counts_ref[...] = jnp.zeros_like(counts_ref)
        loss_ref[...] = jnp.zeros_like(loss_ref)

    x = x_ref[...]
    logits = jnp.dot(x, wrt_ref[...],
                     preferred_element_type=jnp.float32)
    logits = logits + br_ref[...]

    loss_ref[...] = loss_ref[...] + jnp.sum(logits * logits)

    eidx = lax.broadcasted_iota(jnp.int32, logits.shape, 1)
    m1 = jnp.max(logits, axis=1, keepdims=True)
    a1 = jnp.min(jnp.where(logits == m1, eidx, _E), axis=1, keepdims=True)
    neg = jnp.float32(-jnp.inf)
    l2 = jnp.where(eidx == a1, neg, logits)
    m2 = jnp.max(l2, axis=1, keepdims=True)
    a2 = jnp.min(jnp.where(l2 == m2, eidx, _E), axis=1, keepdims=True)

    onehot = ((eidx == a1).astype(jnp.float32)
              + (eidx == a2).astype(jnp.float32))
    counts_ref[...] = counts_ref[...] + jnp.sum(onehot, axis=0, keepdims=True)

    @pl.when(i == _GRID - 1)
    def _fin():
        c = jnp.minimum(counts_ref[...], jnp.float32(_CAPACITY))
        counts_ref[...] = c
        load = c / (jnp.sum(c) + jnp.float32(1e-6))
        bal = jnp.float32(_E) * jnp.sum(load * load)
        z = jnp.float32(_Z_COEFF) * loss_ref[...] / jnp.float32(_N_TOK * _E)
        loss_ref[...] = bal + z


def kernel(x, Wr, br, W1, b1, W2, b2):
    xr = x.reshape(_N_TOK, _D)
    wrt = Wr.T
    brr = br.reshape(1, _E)

    counts2, loss2 = pl.pallas_call(
        _router_body,
        grid=(_GRID,),
        in_specs=[
            pl.BlockSpec((_BLK, _D), lambda i: (i, 0)),
            pl.BlockSpec((_D, _E), lambda i: (0, 0)),
            pl.BlockSpec((1, _E), lambda i: (0, 0)),
        ],
        out_specs=[
            pl.BlockSpec((1, _E), lambda i: (0, 0)),
            pl.BlockSpec((1, 1), lambda i: (0, 0)),
        ],
        out_shape=[
            jax.ShapeDtypeStruct((1, _E), jnp.float32),
            jax.ShapeDtypeStruct((1, 1), jnp.float32),
        ],
    )(xr, wrt, brr)

    output = _sc_zeros().reshape(_B, _T, _D)
    return (output, loss2.reshape(()), counts2.reshape(_E))


# speedup vs baseline: 2.4531x; 2.4531x over previous
import jax
import jax.numpy as jnp
from jax import lax
from jax.experimental import pallas as pl
from jax.experimental.pallas import tpu as pltpu

_B, _T, _D = 4, 2048, 1024
_E = 8
_CAP_F = 1.25
_Z_COEFF = 0.001
_N_TOK = _B * _T
_BLK = 2048
_GRID = _N_TOK // _BLK
_CAPACITY = float(int(_CAP_F * _N_TOK / _E))


def _router_body(x_ref, wrt_ref, br_ref, counts_ref, loss_ref):
    i = pl.program_id(0)

    @pl.when(i == 0)
    def _init():
        counts_ref[...] = jnp.zeros_like(counts_ref)
        loss_ref[...] = jnp.zeros_like(loss_ref)

    x = x_ref[...]
    logits = jnp.dot(x, wrt_ref[...],
                     preferred_element_type=jnp.float32)
    logits = logits + br_ref[...]

    loss_ref[...] = loss_ref[...] + jnp.sum(logits * logits)

    eidx = lax.broadcasted_iota(jnp.int32, logits.shape, 1)
    m1 = jnp.max(logits, axis=1, keepdims=True)
    a1 = jnp.min(jnp.where(logits == m1, eidx, _E), axis=1, keepdims=True)
    neg = jnp.float32(-jnp.inf)
    l2 = jnp.where(eidx == a1, neg, logits)
    m2 = jnp.max(l2, axis=1, keepdims=True)
    a2 = jnp.min(jnp.where(l2 == m2, eidx, _E), axis=1, keepdims=True)

    onehot = ((eidx == a1).astype(jnp.float32)
              + (eidx == a2).astype(jnp.float32))
    counts_ref[...] = counts_ref[...] + jnp.sum(onehot, axis=0, keepdims=True)

    @pl.when(i == _GRID - 1)
    def _fin():
        c = jnp.minimum(counts_ref[...], jnp.float32(_CAPACITY))
        counts_ref[...] = c
        load = c / (jnp.sum(c) + jnp.float32(1e-6))
        bal = jnp.float32(_E) * jnp.sum(load * load)
        z = jnp.float32(_Z_COEFF) * loss_ref[...] / jnp.float32(_N_TOK * _E)
        loss_ref[...] = bal + z


def kernel(x, Wr, br, W1, b1, W2, b2):
    xr = x.reshape(_N_TOK, _D)
    wrt = Wr.T
    brr = br.reshape(1, _E)

    counts2, loss2 = pl.pallas_call(
        _router_body,
        grid=(_GRID,),
        in_specs=[
            pl.BlockSpec((_BLK, _D), lambda i: (i, 0)),
            pl.BlockSpec((_D, _E), lambda i: (0, 0)),
            pl.BlockSpec((1, _E), lambda i: (0, 0)),
        ],
        out_specs=[
            pl.BlockSpec((1, _E), lambda i: (0, 0)),
            pl.BlockSpec((1, 1), lambda i: (0, 0)),
        ],
        out_shape=[
            jax.ShapeDtypeStruct((1, _E), jnp.float32),
            jax.ShapeDtypeStruct((1, 1), jnp.float32),
        ],
    )(xr, wrt, brr)

    output = jnp.zeros((_B, _T, _D), jnp.float32)
    return (output, loss2.reshape(()), counts2.reshape(_E))


# speedup vs baseline: 2.8169x; 1.1483x over previous
import jax
import jax.numpy as jnp
from jax import lax
from jax.experimental import pallas as pl
from jax.experimental.pallas import tpu as pltpu

_B, _T, _D = 4, 2048, 1024
_E = 8
_CAP_F = 1.25
_Z_COEFF = 0.001
_N_TOK = _B * _T
_BLK = 2048
_GRID = _N_TOK // _BLK
_CAPACITY = float(int(_CAP_F * _N_TOK / _E))


def _router_body(x_ref, wr_ref, br_ref, counts_ref, loss_ref, zout_ref,
                 acc_ref, sq_ref):
    i = pl.program_id(0)
    zout_ref[...] = jnp.zeros_like(zout_ref)

    @pl.when(i == 0)
    def _init():
        acc_ref[...] = jnp.zeros_like(acc_ref)
        sq_ref[...] = jnp.zeros_like(sq_ref)

    x = x_ref[...]
    logitsT = lax.dot_general(
        wr_ref[...], x, (((1,), (1,)), ((), ())),
        preferred_element_type=jnp.float32)
    logitsT = logitsT + br_ref[...]

    sq_ref[...] = sq_ref[...] + logitsT * logitsT

    eidx = lax.broadcasted_iota(jnp.int32, (_E, _BLK), 0)
    rank = jnp.zeros((_E, _BLK), jnp.float32)
    for j in range(_E):
        lj = logitsT[j:j + 1, :]
        beats = jnp.where(lj > logitsT, 1.0,
                          jnp.where((lj == logitsT) & (j < eidx), 1.0, 0.0))
        rank = rank + beats
    member = (rank < 1.5).astype(jnp.float32)
    acc_ref[...] = acc_ref[...] + member

    @pl.when(i == _GRID - 1)
    def _fin():
        counts_col = jnp.sum(acc_ref[...], axis=1, keepdims=True)
        c = jnp.minimum(counts_col, jnp.float32(_CAPACITY))
        counts_ref[...] = c
        load = c / (jnp.sum(c) + jnp.float32(1e-6))
        bal = jnp.float32(_E) * jnp.sum(load * load)
        z = jnp.float32(_Z_COEFF) * jnp.sum(sq_ref[...]) / jnp.float32(_N_TOK * _E)
        loss_ref[...] = (bal + z).reshape(1, 1)


def kernel(x, Wr, br, W1, b1, W2, b2):
    xr = x.reshape(_N_TOK, _D)
    brr = br.reshape(_E, 1)

    counts2, loss2, zout = pl.pallas_call(
        _router_body,
        grid=(_GRID,),
        in_specs=[
            pl.BlockSpec((_BLK, _D), lambda i: (i, 0)),
            pl.BlockSpec((_E, _D), lambda i: (0, 0)),
            pl.BlockSpec((_E, 1), lambda i: (0, 0)),
        ],
        out_specs=[
            pl.BlockSpec((_E, 1), lambda i: (0, 0)),
            pl.BlockSpec((1, 1), lambda i: (0, 0)),
            pl.BlockSpec((_BLK, _D), lambda i: (i, 0)),
        ],
        out_shape=[
            jax.ShapeDtypeStruct((_E, 1), jnp.float32),
            jax.ShapeDtypeStruct((1, 1), jnp.float32),
            jax.ShapeDtypeStruct((_N_TOK, _D), jnp.float32),
        ],
        scratch_shapes=[
            pltpu.VMEM((_E, _BLK), jnp.float32),
            pltpu.VMEM((_E, _BLK), jnp.float32),
        ],
    )(xr, Wr, brr)

    return (zout.reshape(_B, _T, _D), loss2.reshape(()), counts2.reshape(_E))
